# baseline (speedup 1.0000x reference)
import functools

import jax
import jax.numpy as jnp
from jax.experimental import pallas as pl
from jax.experimental.pallas import tpu as pltpu


def _agg(x, src, dst, n):
    msgs = jnp.take(x, src, axis=0)
    s = jax.ops.segment_sum(msgs, dst, num_segments=n)
    cnt = jax.ops.segment_sum(jnp.ones((src.shape[0],), x.dtype), dst, num_segments=n)
    return s / jnp.clip(cnt, 1.0, None)[:, None]


def _final_body(agg_ref, h_ref, wl_ref, wr_ref, b_ref, o_ref):
    logits = (
        jnp.dot(agg_ref[...], wl_ref[...], preferred_element_type=jnp.float32)
        + jnp.dot(h_ref[...], wr_ref[...], preferred_element_type=jnp.float32)
        + b_ref[...][None, :]
    )
    ncls = o_ref.shape[1]
    col = jax.lax.broadcasted_iota(jnp.int32, logits.shape, 1)
    masked = jnp.where(col < ncls, logits, -jnp.inf)
    m = jnp.max(masked, axis=1, keepdims=True)
    lse = m + jnp.log(jnp.sum(jnp.where(col < ncls, jnp.exp(masked - m), 0.0),
                              axis=1, keepdims=True))
    o_ref[...] = (logits - lse)[:, :ncls]


def kernel(x, edge_index, Wl0, Wr0, b0, Wl1, Wr1, b1, Wl2, Wr2, b2):
    src = edge_index[0]
    dst = edge_index[1]
    n = x.shape[0]
    h = x
    for (Wl, Wr, b) in ((Wl0, Wr0, b0), (Wl1, Wr1, b1)):
        a = _agg(h, src, dst, n)
        h = jax.nn.relu(a @ Wl + b + h @ Wr)
    a = _agg(h, src, dst, n)
    C = Wl2.shape[1]
    Wl2p = jnp.zeros((Wl2.shape[0], 128), jnp.float32).at[:, :C].set(Wl2)
    Wr2p = jnp.zeros((Wr2.shape[0], 128), jnp.float32).at[:, :C].set(Wr2)
    b2p = jnp.zeros((128,), jnp.float32).at[:C].set(b2)
    bm = 1000
    out = pl.pallas_call(
        _final_body,
        grid=(n // bm,),
        in_specs=[
            pl.BlockSpec((bm, 128), lambda i: (i, 0)),
            pl.BlockSpec((bm, 128), lambda i: (i, 0)),
            pl.BlockSpec((128, 128), lambda i: (0, 0)),
            pl.BlockSpec((128, 128), lambda i: (0, 0)),
            pl.BlockSpec((128,), lambda i: (0,)),
        ],
        out_specs=pl.BlockSpec((bm, C), lambda i: (i, 0)),
        out_shape=jax.ShapeDtypeStruct((n, C), jnp.float32),
    )(a, h, Wl2p, Wr2p, b2p)
    return out


# speedup vs baseline: 9.0539x; 9.0539x over previous
import functools

import jax
import jax.numpy as jnp
from jax import lax
from jax.experimental import pallas as pl
from jax.experimental.pallas import tpu as pltpu
from jax.experimental.pallas import tpu_sc as plsc

N = 10000
NP = 10240
E = 320000
F = 128
NC, NS = 2, 16
NW = NC * NS
EPW = E // NW
B = 50
NB = EPW // B
NBUF = 5
CB = 40
NCH = NB // CB
TPN = NP // NS
ZR = 16

_mesh = plsc.VectorSubcoreMesh(core_axis_name="c", subcore_axis_name="s")


def _sc_agg_body(h_hbm, src_hbm, dst_hbm, acc_hbm,
                 src_idx, dst_idx, rows, zbuf, acc, s0, s1, s2, s3, s4):
    sems = (s0, s1, s2, s3, s4)
    c = lax.axis_index("c")
    s = lax.axis_index("s")
    w = c * NS + s
    base = s * TPN

    def fill_zero(i, carry):
        for j in range(F // 16):
            zbuf[i, pl.ds(16 * j, 16)] = jnp.zeros((16,), jnp.float32)
        return carry

    lax.fori_loop(0, ZR, fill_zero, 0)

    for m in range(TPN // ZR):
        pltpu.sync_copy(zbuf, acc.at[pl.ds(base + ZR * m, ZR)])

    plsc.subcore_barrier()

    def fire(b, k):
        pltpu.async_copy(h_hbm.at[src_idx.at[b]], rows.at[k], sems[k])

    def drain(b, k):
        pltpu.make_async_copy(h_hbm.at[src_idx.at[b]], rows.at[k], sems[k]).wait()

    def scat(b, k):
        pltpu.sync_copy(rows.at[k], acc.at[dst_idx.at[b]], add=True)

    for ci in range(NCH):
        pltpu.sync_copy(src_hbm.at[w].at[pl.ds(ci * CB, CB)], src_idx)
        pltpu.sync_copy(dst_hbm.at[w].at[pl.ds(ci * CB, CB)], dst_idx)
        for k in range(NBUF):
            fire(k, k)

        def group(i, carry):
            t = i * NBUF
            for k in range(NBUF):
                b = t + k
                drain(b, k)
                scat(b, k)

                @pl.when(b + NBUF <= CB - 1)
                def _():
                    fire(b + NBUF, k)
            return carry

        lax.fori_loop(0, CB // NBUF - 1, group, 0)
        for k in range(NBUF):
            b = CB - NBUF + k
            drain(b, k)
            scat(b, k)

    plsc.subcore_barrier()

    for m in range(TPN // ZR):
        sl = pl.ds(base + ZR * m, ZR)
        pltpu.sync_copy(acc.at[sl], acc_hbm.at[c].at[sl])


_sc_agg = pl.kernel(
    _sc_agg_body,
    out_type=[jax.ShapeDtypeStruct((NC, NP, F), jnp.float32)],
    mesh=_mesh,
    scratch_types=[
        pltpu.VMEM((CB, B), jnp.int32),
        pltpu.VMEM((CB, B), jnp.int32),
        pltpu.VMEM((NBUF, B, F), jnp.float32),
        pltpu.VMEM((ZR, F), jnp.float32),
        pltpu.VMEM_SHARED((NP, F), jnp.float32),
        pltpu.SemaphoreType.DMA,
        pltpu.SemaphoreType.DMA,
        pltpu.SemaphoreType.DMA,
        pltpu.SemaphoreType.DMA,
        pltpu.SemaphoreType.DMA,
    ],
    name="sc_segment_sum",
)


def _tc_layer_body(final, parts_ref, cntp_ref, h_ref, wl_ref, wr_ref, b_ref, o_ref):
    p = parts_ref[0] + parts_ref[1]
    cnt_lanes = jnp.sum(cntp_ref[0] + cntp_ref[1], axis=-1)
    inv = 128.0 / jnp.clip(cnt_lanes, 128.0, None)
    agg = p * inv[:, None]
    y = (jnp.dot(agg, wl_ref[...], preferred_element_type=jnp.float32)
         + jnp.dot(h_ref[...], wr_ref[...], preferred_element_type=jnp.float32)
         + b_ref[...][None, :])
    if not final:
        o_ref[...] = jnp.maximum(y, 0.0)
    else:
        ncls = 47
        col = lax.broadcasted_iota(jnp.int32, y.shape, 1)
        valid = col < ncls
        masked = jnp.where(valid, y, -jnp.inf)
        m = jnp.max(masked, axis=1, keepdims=True)
        lse = m + jnp.log(jnp.sum(jnp.where(valid, jnp.exp(masked - m), 0.0),
                                  axis=1, keepdims=True))
        o_ref[...] = y - lse


def _tc_layer(parts, cntp, h, wl, wr, b, final):
    bm = 1000
    return pl.pallas_call(
        functools.partial(_tc_layer_body, final),
        grid=(N // bm,),
        in_specs=[
            pl.BlockSpec((NC, bm, F), lambda i: (0, i, 0)),
            pl.BlockSpec((NC, bm, F), lambda i: (0, i, 0)),
            pl.BlockSpec((bm, F), lambda i: (i, 0)),
            pl.BlockSpec((F, F), lambda i: (0, 0)),
            pl.BlockSpec((F, F), lambda i: (0, 0)),
            pl.BlockSpec((F,), lambda i: (0,)),
        ],
        out_specs=pl.BlockSpec((bm, F), lambda i: (i, 0)),
        out_shape=jax.ShapeDtypeStruct((N, F), jnp.float32),
    )(parts, cntp, h, wl, wr, b)


def kernel(x, edge_index, Wl0, Wr0, b0, Wl1, Wr1, b1, Wl2, Wr2, b2):
    src2 = edge_index[0].astype(jnp.int32).reshape(NW, NB, B)
    dst2 = edge_index[1].astype(jnp.int32).reshape(NW, NB, B)

    C = Wl2.shape[1]
    Wl2p = jnp.zeros((F, F), jnp.float32).at[:, :C].set(Wl2)
    Wr2p = jnp.zeros((F, F), jnp.float32).at[:, :C].set(Wr2)
    b2p = jnp.zeros((F,), jnp.float32).at[:C].set(b2)

    (cntp,) = _sc_agg(jnp.ones((N, F), jnp.float32), dst2, dst2)
    (parts0,) = _sc_agg(x, src2, dst2)
    h1 = _tc_layer(parts0, cntp, x, Wl0, Wr0, b0, final=False)
    (parts1,) = _sc_agg(h1, src2, dst2)
    h2 = _tc_layer(parts1, cntp, h1, Wl1, Wr1, b1, final=False)
    (parts2,) = _sc_agg(h2, src2, dst2)
    out = _tc_layer(parts2, cntp, h2, Wl2p, Wr2p, b2p, final=True)
    return out[:, :C]


# speedup vs baseline: 9.6435x; 1.0651x over previous
import functools

import jax
import jax.numpy as jnp
from jax import lax
from jax.experimental import pallas as pl
from jax.experimental.pallas import tpu as pltpu
from jax.experimental.pallas import tpu_sc as plsc

N = 10000
NP = 10240
E = 320000
F = 128
NC, NS = 2, 16
NW = NC * NS
EPW = E // NW
TPN = NP // NS
ZR = 16

_mesh = plsc.VectorSubcoreMesh(core_axis_name="c", subcore_axis_name="s")


def _sc_agg_body(B, NB, NBUF, NCH, *refs):
    (h_hbm, src_hbm, dst_hbm, acc_hbm, src_idx, dst_idx, rows, zbuf, acc,
     *sems) = refs
    CB = NB // NCH
    c = lax.axis_index("c")
    s = lax.axis_index("s")
    w = c * NS + s
    base = s * TPN

    def fill_zero(i, carry):
        for j in range(F // 16):
            zbuf[i, pl.ds(16 * j, 16)] = jnp.zeros((16,), jnp.float32)
        return carry

    lax.fori_loop(0, ZR, fill_zero, 0)

    for m in range(TPN // ZR):
        pltpu.sync_copy(zbuf, acc.at[pl.ds(base + ZR * m, ZR)])

    plsc.subcore_barrier()

    def fire(b, k):
        pltpu.async_copy(h_hbm.at[src_idx.at[b]], rows.at[k], sems[k])

    def drain(b, k):
        pltpu.make_async_copy(h_hbm.at[src_idx.at[b]], rows.at[k], sems[k]).wait()

    def scat(b, k):
        pltpu.sync_copy(rows.at[k], acc.at[dst_idx.at[b]], add=True)

    for ci in range(NCH):
        if NCH == 1:
            pltpu.sync_copy(src_hbm.at[w], src_idx)
            pltpu.sync_copy(dst_hbm.at[w], dst_idx)
        else:
            pltpu.sync_copy(src_hbm.at[w].at[pl.ds(ci * CB, CB)], src_idx)
            pltpu.sync_copy(dst_hbm.at[w].at[pl.ds(ci * CB, CB)], dst_idx)
        for k in range(NBUF):
            fire(k, k)

        def group(i, carry):
            t = i * NBUF
            for k in range(NBUF):
                b = t + k
                drain(b, k)
                scat(b, k)

                @pl.when(b + NBUF <= CB - 1)
                def _():
                    fire(b + NBUF, k)
            return carry

        lax.fori_loop(0, CB // NBUF - 1, group, 0)
        for k in range(NBUF):
            b = CB - NBUF + k
            drain(b, k)
            scat(b, k)

    plsc.subcore_barrier()

    for m in range(TPN // ZR):
        sl = pl.ds(base + ZR * m, ZR)
        pltpu.sync_copy(acc.at[sl], acc_hbm.at[c].at[sl])


def _make_sc_agg(B, NBUF, NCH):
    NB = EPW // B
    CB = NB // NCH
    return pl.kernel(
        functools.partial(_sc_agg_body, B, NB, NBUF, NCH),
        out_type=[jax.ShapeDtypeStruct((NC, NP, F), jnp.float32)],
        mesh=_mesh,
        scratch_types=[
            pltpu.VMEM((CB, B), jnp.int32),
            pltpu.VMEM((CB, B), jnp.int32),
            pltpu.VMEM((NBUF, B, F), jnp.float32),
            pltpu.VMEM((ZR, F), jnp.float32),
            pltpu.VMEM_SHARED((NP, F), jnp.float32),
        ] + [pltpu.SemaphoreType.DMA] * NBUF,
        name="sc_segment_sum",
    )


_sc_agg = _make_sc_agg(50, 5, 5)
_CB = 80


def _sc_cnt_body(dst_hbm, ones_hbm, cnt_hbm, dst_idx, ones, zbuf, cnt, sem):
    NB = EPW // _CB
    c = lax.axis_index("c")
    s = lax.axis_index("s")
    w = c * NS + s
    base = s * TPN

    def fill_zero(i, carry):
        for j in range(F // 16):
            zbuf[i, pl.ds(16 * j, 16)] = jnp.zeros((16,), jnp.float32)
        return carry

    lax.fori_loop(0, ZR, fill_zero, 0)
    for m in range(TPN // ZR):
        pltpu.sync_copy(zbuf, cnt.at[pl.ds(base + ZR * m, ZR)])

    pltpu.sync_copy(ones_hbm, ones)
    pltpu.sync_copy(dst_hbm.at[w], dst_idx)
    plsc.subcore_barrier()

    def group(i, carry):
        t = i * 5
        for k in range(5):
            pltpu.async_copy(ones, cnt.at[dst_idx.at[t + k]], sem, add=True)
        for k in range(5):
            pltpu.make_async_copy(ones, cnt.at[dst_idx.at[t + k]], sem).wait()
        return carry

    lax.fori_loop(0, NB // 5, group, 0)
    plsc.subcore_barrier()

    for m in range(TPN // ZR):
        sl = pl.ds(base + ZR * m, ZR)
        pltpu.sync_copy(cnt.at[sl], cnt_hbm.at[c].at[sl])


_sc_cnt = pl.kernel(
    _sc_cnt_body,
    out_type=[jax.ShapeDtypeStruct((NC, NP, F), jnp.float32)],
    mesh=_mesh,
    scratch_types=[
        pltpu.VMEM((EPW // _CB, _CB), jnp.int32),
        pltpu.VMEM((_CB, F), jnp.float32),
        pltpu.VMEM((ZR, F), jnp.float32),
        pltpu.VMEM_SHARED((NP, F), jnp.float32),
        pltpu.SemaphoreType.DMA,
    ],
    name="sc_segment_cnt",
)


def _tc_layer_body(final, parts_ref, cntp_ref, h_ref, wl_ref, wr_ref, b_ref,
                   o_ref):
    p = parts_ref[0] + parts_ref[1]
    cnt_lanes = jnp.sum(cntp_ref[0] + cntp_ref[1], axis=-1)
    inv = 128.0 / jnp.clip(cnt_lanes, 128.0, None)
    agg = p * inv[:, None]
    y = (jnp.dot(agg, wl_ref[...], preferred_element_type=jnp.float32)
         + jnp.dot(h_ref[...], wr_ref[...], preferred_element_type=jnp.float32)
         + b_ref[...][None, :])
    if not final:
        o_ref[...] = jnp.maximum(y, 0.0)
    else:
        ncls = 47
        col = lax.broadcasted_iota(jnp.int32, y.shape, 1)
        valid = col < ncls
        masked = jnp.where(valid, y, -jnp.inf)
        m = jnp.max(masked, axis=1, keepdims=True)
        lse = m + jnp.log(jnp.sum(jnp.where(valid, jnp.exp(masked - m), 0.0),
                                  axis=1, keepdims=True))
        o_ref[...] = y - lse


def _tc_layer(parts, cntp, h, wl, wr, b, final):
    bm = 1000
    return pl.pallas_call(
        functools.partial(_tc_layer_body, final),
        grid=(N // bm,),
        in_specs=[
            pl.BlockSpec((NC, bm, F), lambda i: (0, i, 0)),
            pl.BlockSpec((NC, bm, F), lambda i: (0, i, 0)),
            pl.BlockSpec((bm, F), lambda i: (i, 0)),
            pl.BlockSpec((F, F), lambda i: (0, 0)),
            pl.BlockSpec((F, F), lambda i: (0, 0)),
            pl.BlockSpec((F,), lambda i: (0,)),
        ],
        out_specs=pl.BlockSpec((bm, F), lambda i: (i, 0)),
        out_shape=jax.ShapeDtypeStruct((N, F), jnp.float32),
    )(parts, cntp, h, wl, wr, b)


def kernel(x, edge_index, Wl0, Wr0, b0, Wl1, Wr1, b1, Wl2, Wr2, b2):
    src = edge_index[0].astype(jnp.int32)
    dst = edge_index[1].astype(jnp.int32)
    B0 = 50
    src2 = src.reshape(NW, EPW // B0, B0)
    dst2 = dst.reshape(NW, EPW // B0, B0)
    dst2b = dst.reshape(NW, EPW // _CB, _CB)

    C = Wl2.shape[1]
    Wl2p = jnp.zeros((F, F), jnp.float32).at[:, :C].set(Wl2)
    Wr2p = jnp.zeros((F, F), jnp.float32).at[:, :C].set(Wr2)
    b2p = jnp.zeros((F,), jnp.float32).at[:C].set(b2)

    (cntp,) = _sc_cnt(dst2b, jnp.ones((_CB, F), jnp.float32))
    (parts0,) = _sc_agg(x, src2, dst2)
    h1 = _tc_layer(parts0, cntp, x, Wl0, Wr0, b0, final=False)
    (parts1,) = _sc_agg(h1, src2, dst2)
    h2 = _tc_layer(parts1, cntp, h1, Wl1, Wr1, b1, final=False)
    (parts2,) = _sc_agg(h2, src2, dst2)
    out = _tc_layer(parts2, cntp, h2, Wl2p, Wr2p, b2p, final=True)
    return out[:, :C]


# speedup vs baseline: 10.6216x; 1.1014x over previous
import functools

import jax
import jax.numpy as jnp
from jax import lax
from jax.experimental import pallas as pl
from jax.experimental.pallas import tpu as pltpu
from jax.experimental.pallas import tpu_sc as plsc

N = 10000
NP = 10240
E = 320000
F = 128
NC, NS = 2, 16
NW = NC * NS
EPW = E // NW
TPN = NP // NS

_mesh = plsc.VectorSubcoreMesh(core_axis_name="c", subcore_axis_name="s")


def _sc_agg_body(B, NB, NBUF, NCH, *refs):
    (h_hbm, src_hbm, dst_hbm, zeros_hbm, acc_hbm, src_idx, dst_idx, rows, acc,
     *sems) = refs
    gsems = sems[:NBUF]
    ssems = sems[NBUF:]
    CB = NB // NCH
    c = lax.axis_index("c")
    s = lax.axis_index("s")
    w = c * NS + s
    base = s * TPN

    pltpu.sync_copy(zeros_hbm.at[pl.ds(base, TPN)], acc.at[pl.ds(base, TPN)])
    plsc.subcore_barrier()

    def fire_g(b, k):
        pltpu.async_copy(h_hbm.at[src_idx.at[b]], rows.at[k], gsems[k])

    def drain_g(b, k):
        pltpu.make_async_copy(h_hbm.at[src_idx.at[b]], rows.at[k],
                              gsems[k]).wait()

    def fire_s(b, k):
        pltpu.async_copy(rows.at[k], acc.at[dst_idx.at[b]], ssems[k], add=True)

    def drain_s(b, k):
        pltpu.make_async_copy(rows.at[k], acc.at[dst_idx.at[b]],
                              ssems[k]).wait()

    for ci in range(NCH):
        if NCH == 1:
            pltpu.sync_copy(src_hbm.at[w], src_idx)
            pltpu.sync_copy(dst_hbm.at[w], dst_idx)
        else:
            pltpu.sync_copy(src_hbm.at[w].at[pl.ds(ci * CB, CB)], src_idx)
            pltpu.sync_copy(dst_hbm.at[w].at[pl.ds(ci * CB, CB)], dst_idx)
        for k in range(3):
            fire_g(k, k)

        def group(i, carry):
            t = i * NBUF
            for k in range(NBUF):
                b = t + k
                kn = (k + 3) % NBUF
                drain_g(b, k)
                fire_s(b, k)

                @pl.when(b >= 2)
                def _():
                    drain_s(b - 2, kn)

                @pl.when(b + 3 <= CB - 1)
                def _():
                    fire_g(b + 3, kn)
            return carry

        lax.fori_loop(0, CB // NBUF, group, 0)
        drain_s(CB - 2, (CB - 2) % NBUF)
        drain_s(CB - 1, (CB - 1) % NBUF)

    plsc.subcore_barrier()
    pltpu.sync_copy(acc.at[pl.ds(base, TPN)],
                    acc_hbm.at[c].at[pl.ds(base, TPN)])


def _make_sc_agg(B, NBUF, NCH):
    NB = EPW // B
    CB = NB // NCH
    return pl.kernel(
        functools.partial(_sc_agg_body, B, NB, NBUF, NCH),
        out_type=[jax.ShapeDtypeStruct((NC, NP, F), jnp.float32)],
        mesh=_mesh,
        scratch_types=[
            pltpu.VMEM((CB, B), jnp.int32),
            pltpu.VMEM((CB, B), jnp.int32),
            pltpu.VMEM((NBUF, B, F), jnp.float32),
            pltpu.VMEM_SHARED((NP, F), jnp.float32),
        ] + [pltpu.SemaphoreType.DMA] * (2 * NBUF),
        name="sc_segment_sum",
    )


_sc_agg = _make_sc_agg(50, 5, 5)
_CB = 80


def _sc_cnt_body(dst_hbm, ones_hbm, zeros_hbm, cnt_hbm, dst_idx, ones, cnt, sem):
    NB = EPW // _CB
    c = lax.axis_index("c")
    s = lax.axis_index("s")
    w = c * NS + s
    base = s * TPN

    pltpu.sync_copy(zeros_hbm.at[pl.ds(base, TPN)], cnt.at[pl.ds(base, TPN)])
    pltpu.sync_copy(ones_hbm, ones)
    pltpu.sync_copy(dst_hbm.at[w], dst_idx)
    plsc.subcore_barrier()

    def group(i, carry):
        t = i * 5
        for k in range(5):
            pltpu.async_copy(ones, cnt.at[dst_idx.at[t + k]], sem, add=True)
        for k in range(5):
            pltpu.make_async_copy(ones, cnt.at[dst_idx.at[t + k]], sem).wait()
        return carry

    lax.fori_loop(0, NB // 5, group, 0)
    plsc.subcore_barrier()

    pltpu.sync_copy(cnt.at[pl.ds(base, TPN)],
                    cnt_hbm.at[c].at[pl.ds(base, TPN)])


_sc_cnt = pl.kernel(
    _sc_cnt_body,
    out_type=[jax.ShapeDtypeStruct((NC, NP, F), jnp.float32)],
    mesh=_mesh,
    scratch_types=[
        pltpu.VMEM((EPW // _CB, _CB), jnp.int32),
        pltpu.VMEM((_CB, F), jnp.float32),
        pltpu.VMEM_SHARED((NP, F), jnp.float32),
        pltpu.SemaphoreType.DMA,
    ],
    name="sc_segment_cnt",
)


def _tc_layer_body(final, parts_ref, cntp_ref, h_ref, wl_ref, wr_ref, b_ref,
                   o_ref):
    p = parts_ref[0] + parts_ref[1]
    cnt_lanes = jnp.sum(cntp_ref[0] + cntp_ref[1], axis=-1)
    inv = 128.0 / jnp.clip(cnt_lanes, 128.0, None)
    agg = p * inv[:, None]
    y = (jnp.dot(agg, wl_ref[...], preferred_element_type=jnp.float32)
         + jnp.dot(h_ref[...], wr_ref[...], preferred_element_type=jnp.float32)
         + b_ref[...][None, :])
    if not final:
        o_ref[...] = jnp.maximum(y, 0.0)
    else:
        ncls = o_ref.shape[1]
        col = lax.broadcasted_iota(jnp.int32, y.shape, 1)
        valid = col < ncls
        masked = jnp.where(valid, y, -jnp.inf)
        m = jnp.max(masked, axis=1, keepdims=True)
        lse = m + jnp.log(jnp.sum(jnp.where(valid, jnp.exp(masked - m), 0.0),
                                  axis=1, keepdims=True))
        o_ref[...] = (y - lse)[:, :ncls]


def _tc_layer(parts, cntp, h, wl, wr, b, final, ncols=F):
    bm = 1000
    return pl.pallas_call(
        functools.partial(_tc_layer_body, final),
        grid=(N // bm,),
        in_specs=[
            pl.BlockSpec((NC, bm, F), lambda i: (0, i, 0)),
            pl.BlockSpec((NC, bm, F), lambda i: (0, i, 0)),
            pl.BlockSpec((bm, F), lambda i: (i, 0)),
            pl.BlockSpec((F, F), lambda i: (0, 0)),
            pl.BlockSpec((F, F), lambda i: (0, 0)),
            pl.BlockSpec((F,), lambda i: (0,)),
        ],
        out_specs=pl.BlockSpec((bm, ncols), lambda i: (i, 0)),
        out_shape=jax.ShapeDtypeStruct((N, ncols), jnp.float32),
    )(parts, cntp, h, wl, wr, b)


def kernel(x, edge_index, Wl0, Wr0, b0, Wl1, Wr1, b1, Wl2, Wr2, b2):
    src = edge_index[0].astype(jnp.int32)
    dst = edge_index[1].astype(jnp.int32)
    B0 = 50
    src2 = src.reshape(NW, EPW // B0, B0)
    dst2 = dst.reshape(NW, EPW // B0, B0)
    dst2b = dst.reshape(NW, EPW // _CB, _CB)

    C = Wl2.shape[1]
    Wl2p = jnp.zeros((F, F), jnp.float32).at[:, :C].set(Wl2)
    Wr2p = jnp.zeros((F, F), jnp.float32).at[:, :C].set(Wr2)
    b2p = jnp.zeros((F,), jnp.float32).at[:C].set(b2)

    zeros = jnp.zeros((NP, F), jnp.float32)
    (cntp,) = _sc_cnt(dst2b, jnp.ones((_CB, F), jnp.float32), zeros)
    (parts0,) = _sc_agg(x, src2, dst2, zeros)
    h1 = _tc_layer(parts0, cntp, x, Wl0, Wr0, b0, final=False)
    (parts1,) = _sc_agg(h1, src2, dst2, zeros)
    h2 = _tc_layer(parts1, cntp, h1, Wl1, Wr1, b1, final=False)
    (parts2,) = _sc_agg(h2, src2, dst2, zeros)
    return _tc_layer(parts2, cntp, h2, Wl2p, Wr2p, b2p, final=True, ncols=C)
